# baseline (speedup 1.0000x reference)
import functools

import jax
import jax.numpy as jnp
from jax import lax
from jax.experimental import pallas as pl
from jax.experimental.pallas import tpu as pltpu
from jax.experimental.pallas import tpu_sc as plsc

N = 100000
D = 128
C = 200
NCHUNKS = N // C
NW = 32
NC_BASE = NCHUNKS // NW
EXTRA = NCHUNKS - NC_BASE * NW
MAXC = NC_BASE + 1
NBUF = 4
LOOKAHEAD = 2


def kernel(x, embedding_weight):
    idx = x.astype(jnp.int32)
    mesh = plsc.VectorSubcoreMesh(core_axis_name="c", subcore_axis_name="s")

    @functools.partial(
        pl.kernel,
        mesh=mesh,
        out_type=jax.ShapeDtypeStruct((N, D), jnp.float32),
        scratch_types=[
            pltpu.VMEM((MAXC * C,), jnp.int32),
            pltpu.VMEM((NBUF, C, D), jnp.float32),
            pltpu.SemaphoreType.DMA,
            pltpu.SemaphoreType.DMA,
        ],
    )
    def emb(idx_hbm, table_hbm, out_hbm, idx_v, rows_v, sem_g, sem_w):
        w = lax.axis_index("s") * 2 + lax.axis_index("c")
        nc = NC_BASE + jnp.where(w < EXTRA, 1, 0)
        s0 = NC_BASE * w + jnp.minimum(w, EXTRA)
        base = pl.multiple_of(s0 * C, 8)

        pltpu.sync_copy(idx_hbm.at[pl.ds(base, NC_BASE * C)],
                        idx_v.at[pl.ds(0, NC_BASE * C)])

        @pl.when(nc > NC_BASE)
        def _():
            pltpu.sync_copy(
                idx_hbm.at[pl.ds(pl.multiple_of((s0 + NC_BASE) * C, 8), C)],
                idx_v.at[pl.ds(NC_BASE * C, C)])

        def chunk_idx(j):
            return idx_v.at[pl.ds(pl.multiple_of(j * C, 8), C)]

        def issue_gather(j):
            pltpu.async_copy(table_hbm.at[chunk_idx(j)],
                             rows_v.at[j % NBUF], sem_g)

        def wait_gather(j):
            pltpu.make_async_copy(table_hbm.at[chunk_idx(j)],
                                  rows_v.at[j % NBUF], sem_g).wait()

        def issue_wb(j):
            pltpu.async_copy(rows_v.at[j % NBUF],
                             out_hbm.at[pl.ds((s0 + j) * C, C)], sem_w)

        def wait_one_wb():
            pltpu.make_async_copy(rows_v.at[0],
                                  out_hbm.at[pl.ds(s0 * C, C)], sem_w).wait()

        for j in range(LOOKAHEAD):
            issue_gather(j)

        def body(j, carry):
            @pl.when(j < nc)
            def _():
                @pl.when(j + LOOKAHEAD < nc)
                def _():
                    @pl.when(j + LOOKAHEAD >= NBUF)
                    def _():
                        wait_one_wb()

                    issue_gather(j + LOOKAHEAD)

                wait_gather(j)
                issue_wb(j)

            return carry

        lax.fori_loop(0, MAXC, body, 0)

        for _ in range(NBUF):
            wait_one_wb()

    return emb(idx, embedding_weight)


# speedup vs baseline: 1.9993x; 1.9993x over previous
import functools

import jax
import jax.numpy as jnp
from jax import lax
from jax.experimental import pallas as pl
from jax.experimental.pallas import tpu as pltpu
from jax.experimental.pallas import tpu_sc as plsc

N = 100000
D = 128
V = 1000
C = 200
NCHUNKS = N // C
NW = 32
NC_BASE = NCHUNKS // NW
EXTRA = NCHUNKS - NC_BASE * NW
MAXC = NC_BASE + 1
NBUF = 4
LOOKAHEAD = 2


def kernel(x, embedding_weight):
    idx = x.astype(jnp.int32)
    mesh = plsc.VectorSubcoreMesh(core_axis_name="c", subcore_axis_name="s")

    @functools.partial(
        pl.kernel,
        mesh=mesh,
        out_type=jax.ShapeDtypeStruct((N, D), jnp.float32),
        scratch_types=[
            pltpu.VMEM((MAXC * C,), jnp.int32),
            pltpu.VMEM((NBUF, C, D), jnp.float32),
            pltpu.VMEM_SHARED((V, D), jnp.float32),
            pltpu.SemaphoreType.DMA,
            pltpu.SemaphoreType.DMA,
        ],
    )
    def emb(idx_hbm, table_hbm, out_hbm, idx_v, rows_v, table_sp, sem_g, sem_w):
        cid = lax.axis_index("c")
        sid = lax.axis_index("s")
        w = sid * 2 + cid
        nc = NC_BASE + jnp.where(w < EXTRA, 1, 0)
        s0 = NC_BASE * w + jnp.minimum(w, EXTRA)
        base = pl.multiple_of(s0 * C, 8)

        @pl.when(sid == 0)
        def _():
            pltpu.sync_copy(table_hbm, table_sp)

        pltpu.sync_copy(idx_hbm.at[pl.ds(base, NC_BASE * C)],
                        idx_v.at[pl.ds(0, NC_BASE * C)])

        @pl.when(nc > NC_BASE)
        def _():
            pltpu.sync_copy(
                idx_hbm.at[pl.ds(pl.multiple_of((s0 + NC_BASE) * C, 8), C)],
                idx_v.at[pl.ds(NC_BASE * C, C)])

        plsc.subcore_barrier()

        def chunk_idx(j):
            return idx_v.at[pl.ds(pl.multiple_of(j * C, 8), C)]

        def issue_gather(j):
            pltpu.async_copy(table_sp.at[chunk_idx(j)],
                             rows_v.at[j % NBUF], sem_g)

        def wait_gather(j):
            pltpu.make_async_copy(table_sp.at[chunk_idx(j)],
                                  rows_v.at[j % NBUF], sem_g).wait()

        def issue_wb(j):
            pltpu.async_copy(rows_v.at[j % NBUF],
                             out_hbm.at[pl.ds((s0 + j) * C, C)], sem_w)

        def wait_one_wb():
            pltpu.make_async_copy(rows_v.at[0],
                                  out_hbm.at[pl.ds(s0 * C, C)], sem_w).wait()

        for j in range(LOOKAHEAD):
            issue_gather(j)

        def body(j, carry):
            @pl.when(j < nc)
            def _():
                @pl.when(j + LOOKAHEAD < nc)
                def _():
                    @pl.when(j + LOOKAHEAD >= NBUF)
                    def _():
                        wait_one_wb()

                    issue_gather(j + LOOKAHEAD)

                wait_gather(j)
                issue_wb(j)

            return carry

        lax.fori_loop(0, MAXC, body, 0)

        for _ in range(NBUF):
            wait_one_wb()

    return emb(idx, embedding_weight)
